# baseline (speedup 1.0000x reference)
import functools

import jax
import jax.numpy as jnp
from jax import lax
from jax.experimental import pallas as pl
from jax.experimental.pallas import tpu as pltpu
from jax.experimental.pallas import tpu_sc as plsc

_VOCAB = 100000
_DIM = 128
_NTOK = 16384

_NC = 2
_NS = 16
_NW = _NC * _NS
_BPW = _NTOK // _NW
_CHUNK = 128
_NCH = _BPW // _CHUNK
_LANES = 16
_NV = _DIM // _LANES


@functools.partial(
    pl.kernel,
    out_type=[
        jax.ShapeDtypeStruct((_NW, _DIM), jnp.float32),
        jax.ShapeDtypeStruct((1, _DIM), jnp.float32),
    ],
    mesh=plsc.VectorSubcoreMesh(core_axis_name="c", subcore_axis_name="s"),
    scratch_types=[
        pltpu.VMEM((_NCH, _CHUNK), jnp.int32),
        pltpu.VMEM((_BPW, _DIM), jnp.float32),
        pltpu.VMEM((_DIM,), jnp.float32),
        pltpu.VMEM((1,), jnp.int32),
        pltpu.VMEM((1, _DIM), jnp.float32),
        pltpu.SemaphoreType.DMA,
        pltpu.SemaphoreType.DMA,
    ],
)
def _sc_gather_sum(idx_hbm, main_hbm, table_hbm, partials_out, main_out,
                   idx_v, rows_v, acc_v, midx_v, mrow_v, sem, msem):
    wid = lax.axis_index("s") * _NC + lax.axis_index("c")
    base = wid * _BPW
    for c in range(_NCH):
        pltpu.sync_copy(idx_hbm.at[pl.ds(base + c * _CHUNK, _CHUNK)], idx_v.at[c])

    for c in range(_NCH):
        pltpu.async_copy(table_hbm.at[idx_v.at[c]],
                         rows_v.at[pl.ds(c * _CHUNK, _CHUNK)], sem)

    @pl.when(wid == 0)
    def _():
        pltpu.sync_copy(main_hbm, midx_v)
        pltpu.async_copy(table_hbm.at[midx_v], mrow_v, msem)

    _UNROLL = 16
    acc0 = tuple(jnp.zeros((_LANES,), jnp.float32) for _ in range(_NV))

    def chunk_body(c, a):
        pltpu.make_async_copy(table_hbm.at[idx_v.at[0]],
                              rows_v.at[pl.ds(0, _CHUNK)], sem).wait()

        def body(i, a):
            r0 = c * _CHUNK + i * _UNROLL
            for u in range(_UNROLL):
                a = tuple(a[v] + rows_v[r0 + u, pl.ds(v * _LANES, _LANES)]
                          for v in range(_NV))
            return a

        return lax.fori_loop(0, _CHUNK // _UNROLL, body, a)

    acc = lax.fori_loop(0, _NCH, chunk_body, acc0)

    for v in range(_NV):
        acc_v[pl.ds(v * _LANES, _LANES)] = acc[v]
    pltpu.sync_copy(acc_v, partials_out.at[wid])

    @pl.when(wid == 0)
    def _():
        pltpu.make_async_copy(table_hbm.at[midx_v], mrow_v, msem).wait()
        pltpu.sync_copy(mrow_v, main_out)


def _tc_finish_body(partials_ref, mrow_ref, fcw_ref, fcb_ref, out_ref):
    s = jnp.sum(partials_ref[...], axis=0, keepdims=True)
    combined = s * (0.5 / _NTOK) + mrow_ref[...] * 0.5
    out_ref[...] = lax.dot_general(
        combined, fcw_ref[...], (((1,), (1,)), ((), ())),
        preferred_element_type=jnp.float32,
    ) + fcb_ref[...]


def kernel(ingredient_tokens, main_token, emb_table, fc_w, fc_b):
    partials, main_row = _sc_gather_sum(ingredient_tokens, main_token, emb_table)
    out = pl.pallas_call(
        _tc_finish_body,
        out_shape=jax.ShapeDtypeStruct((1, _DIM), jnp.float32),
    )(partials, main_row, fc_w, fc_b.reshape(1, _DIM))
    return out


# speedup vs baseline: 1.0193x; 1.0193x over previous
import functools

import jax
import jax.numpy as jnp
from jax import lax
from jax.experimental import pallas as pl
from jax.experimental.pallas import tpu as pltpu
from jax.experimental.pallas import tpu_sc as plsc

_VOCAB = 100000
_DIM = 128
_NTOK = 16384

_NC = 2
_NS = 16
_NW = _NC * _NS
_BPW = _NTOK // _NW
_CHUNK = 128
_NCH = _BPW // _CHUNK
_LANES = 16
_NV = _DIM // _LANES


@functools.partial(
    pl.kernel,
    out_type=[
        jax.ShapeDtypeStruct((_NW, _DIM), jnp.float32),
        jax.ShapeDtypeStruct((1, _DIM), jnp.float32),
    ],
    mesh=plsc.VectorSubcoreMesh(core_axis_name="c", subcore_axis_name="s"),
    scratch_types=[
        pltpu.VMEM((_BPW,), jnp.int32),
        pltpu.VMEM((_BPW, _DIM), jnp.float32),
        pltpu.VMEM((_DIM,), jnp.float32),
        pltpu.VMEM((1,), jnp.int32),
        pltpu.VMEM((1, _DIM), jnp.float32),
        pltpu.SemaphoreType.DMA((_NCH,)),
        pltpu.SemaphoreType.DMA,
    ],
)
def _sc_gather_sum(idx_hbm, main_hbm, table_hbm, partials_out, main_out,
                   idx_v, rows_v, acc_v, midx_v, mrow_v, sems, msem):
    wid = lax.axis_index("s") * _NC + lax.axis_index("c")
    base = wid * _BPW
    pltpu.sync_copy(idx_hbm.at[pl.ds(base, _BPW)], idx_v)

    for c in range(_NCH):
        pltpu.async_copy(table_hbm.at[idx_v.at[pl.ds(c * _CHUNK, _CHUNK)]],
                         rows_v.at[pl.ds(c * _CHUNK, _CHUNK)], sems.at[c])

    @pl.when(wid == 0)
    def _():
        pltpu.sync_copy(main_hbm, midx_v)
        pltpu.async_copy(table_hbm.at[midx_v], mrow_v, msem)

    _UNROLL = 8
    acc0 = tuple(jnp.zeros((_LANES,), jnp.float32) for _ in range(_NV))

    def chunk_body(c, a):
        pltpu.make_async_copy(table_hbm.at[idx_v.at[pl.ds(0, _CHUNK)]],
                              rows_v.at[pl.ds(0, _CHUNK)], sems.at[c]).wait()

        def body(i, a):
            r0 = c * _CHUNK + i * _UNROLL
            for u in range(_UNROLL):
                a = tuple(a[v] + rows_v[r0 + u, pl.ds(v * _LANES, _LANES)]
                          for v in range(_NV))
            return a

        return lax.fori_loop(0, _CHUNK // _UNROLL, body, a)

    acc = lax.fori_loop(0, _NCH, chunk_body, acc0)

    for v in range(_NV):
        acc_v[pl.ds(v * _LANES, _LANES)] = acc[v]
    pltpu.sync_copy(acc_v, partials_out.at[wid])

    @pl.when(wid == 0)
    def _():
        pltpu.make_async_copy(table_hbm.at[midx_v], mrow_v, msem).wait()
        pltpu.sync_copy(mrow_v, main_out)


def _tc_finish_body(partials_ref, mrow_ref, fcw_ref, fcb_ref, out_ref):
    s = jnp.sum(partials_ref[...], axis=0, keepdims=True)
    combined = s * (0.5 / _NTOK) + mrow_ref[...] * 0.5
    out_ref[...] = lax.dot_general(
        combined, fcw_ref[...], (((1,), (1,)), ((), ())),
        preferred_element_type=jnp.float32,
    ) + fcb_ref[...]


def kernel(ingredient_tokens, main_token, emb_table, fc_w, fc_b):
    partials, main_row = _sc_gather_sum(ingredient_tokens, main_token, emb_table)
    out = pl.pallas_call(
        _tc_finish_body,
        out_shape=jax.ShapeDtypeStruct((1, _DIM), jnp.float32),
    )(partials, main_row, fc_w, fc_b.reshape(1, _DIM))
    return out


# speedup vs baseline: 1.0255x; 1.0061x over previous
import functools

import jax
import jax.numpy as jnp
from jax import lax
from jax.experimental import pallas as pl
from jax.experimental.pallas import tpu as pltpu
from jax.experimental.pallas import tpu_sc as plsc

_VOCAB = 100000
_DIM = 128
_NTOK = 16384

_NC = 2
_NS = 16
_NW = _NC * _NS
_BPW = _NTOK // _NW
_CHUNK = 128
_NCH = _BPW // _CHUNK
_LANES = 16
_NV = _DIM // _LANES


@functools.partial(
    pl.kernel,
    out_type=[
        jax.ShapeDtypeStruct((_NW, _DIM), jnp.float32),
        jax.ShapeDtypeStruct((1, _DIM), jnp.float32),
    ],
    mesh=plsc.VectorSubcoreMesh(core_axis_name="c", subcore_axis_name="s"),
    scratch_types=[
        pltpu.VMEM((_NCH, _CHUNK), jnp.int32),
        pltpu.VMEM((_BPW, _DIM), jnp.float32),
        pltpu.VMEM((_DIM,), jnp.float32),
        pltpu.VMEM((1,), jnp.int32),
        pltpu.VMEM((1, _DIM), jnp.float32),
        pltpu.SemaphoreType.DMA((_NCH,)),
        pltpu.SemaphoreType.DMA,
    ],
)
def _sc_gather_sum(idx_hbm, main_hbm, table_hbm, partials_out, main_out,
                   idx_v, rows_v, acc_v, midx_v, mrow_v, sems, msem):
    wid = lax.axis_index("s") * _NC + lax.axis_index("c")
    base = wid * _BPW
    for c in range(_NCH):
        pltpu.async_copy(idx_hbm.at[pl.ds(base + c * _CHUNK, _CHUNK)],
                         idx_v.at[c], msem)
    for c in range(_NCH):
        pltpu.make_async_copy(idx_hbm.at[pl.ds(base, _CHUNK)],
                              idx_v.at[0], msem).wait()

    for c in range(_NCH):
        pltpu.async_copy(table_hbm.at[idx_v.at[c]],
                         rows_v.at[pl.ds(c * _CHUNK, _CHUNK)], sems.at[c])

    @pl.when(wid == 0)
    def _():
        pltpu.sync_copy(main_hbm, midx_v)
        pltpu.async_copy(table_hbm.at[midx_v], mrow_v, msem)

    _UNROLL = 8
    acc0 = tuple(jnp.zeros((_LANES,), jnp.float32) for _ in range(_NV))

    def chunk_body(c, a):
        pltpu.make_async_copy(table_hbm.at[idx_v.at[0]],
                              rows_v.at[pl.ds(0, _CHUNK)], sems.at[c]).wait()

        def body(i, a):
            r0 = c * _CHUNK + i * _UNROLL
            for u in range(_UNROLL):
                a = tuple(a[v] + rows_v[r0 + u, pl.ds(v * _LANES, _LANES)]
                          for v in range(_NV))
            return a

        return lax.fori_loop(0, _CHUNK // _UNROLL, body, a)

    acc = lax.fori_loop(0, _NCH, chunk_body, acc0)

    for v in range(_NV):
        acc_v[pl.ds(v * _LANES, _LANES)] = acc[v]
    pltpu.sync_copy(acc_v, partials_out.at[wid])

    @pl.when(wid == 0)
    def _():
        pltpu.make_async_copy(table_hbm.at[midx_v], mrow_v, msem).wait()
        pltpu.sync_copy(mrow_v, main_out)


def _tc_finish_body(partials_ref, mrow_ref, fcw_ref, fcb_ref, out_ref):
    s = jnp.sum(partials_ref[...], axis=0, keepdims=True)
    combined = s * (0.5 / _NTOK) + mrow_ref[...] * 0.5
    out_ref[...] = lax.dot_general(
        combined, fcw_ref[...], (((1,), (1,)), ((), ())),
        preferred_element_type=jnp.float32,
    ) + fcb_ref[...]


def kernel(ingredient_tokens, main_token, emb_table, fc_w, fc_b):
    partials, main_row = _sc_gather_sum(ingredient_tokens, main_token, emb_table)
    out = pl.pallas_call(
        _tc_finish_body,
        out_shape=jax.ShapeDtypeStruct((1, _DIM), jnp.float32),
    )(partials, main_row, fc_w, fc_b.reshape(1, _DIM))
    return out


# speedup vs baseline: 1.0477x; 1.0216x over previous
import functools

import jax
import jax.numpy as jnp
from jax import lax
from jax.experimental import pallas as pl
from jax.experimental.pallas import tpu as pltpu
from jax.experimental.pallas import tpu_sc as plsc

_VOCAB = 100000
_DIM = 128
_NTOK = 16384

_NC = 2
_NS = 16
_NW = _NC * _NS
_BPW = _NTOK // _NW
_CHUNK = 64
_NCH = _BPW // _CHUNK
_LANES = 16
_NV = _DIM // _LANES


@functools.partial(
    pl.kernel,
    out_type=[
        jax.ShapeDtypeStruct((_NW, _DIM), jnp.float32),
        jax.ShapeDtypeStruct((1, _DIM), jnp.float32),
    ],
    mesh=plsc.VectorSubcoreMesh(core_axis_name="c", subcore_axis_name="s"),
    scratch_types=[
        pltpu.VMEM((_NCH, _CHUNK), jnp.int32),
        pltpu.VMEM((_BPW, _DIM), jnp.float32),
        pltpu.VMEM((_DIM,), jnp.float32),
        pltpu.VMEM((1,), jnp.int32),
        pltpu.VMEM((1, _DIM), jnp.float32),
        pltpu.SemaphoreType.DMA((_NCH,)),
        pltpu.SemaphoreType.DMA((_NCH,)),
        pltpu.SemaphoreType.DMA,
    ],
)
def _sc_gather_sum(idx_hbm, main_hbm, table_hbm, partials_out, main_out,
                   idx_v, rows_v, acc_v, midx_v, mrow_v, isems, sems, msem):
    wid = lax.axis_index("s") * _NC + lax.axis_index("c")
    base = wid * _BPW
    for c in range(_NCH):
        pltpu.async_copy(idx_hbm.at[pl.ds(base + c * _CHUNK, _CHUNK)],
                         idx_v.at[c], isems.at[c])
    for c in range(_NCH):
        pltpu.make_async_copy(idx_hbm.at[pl.ds(base, _CHUNK)],
                              idx_v.at[c], isems.at[c]).wait()
        pltpu.async_copy(table_hbm.at[idx_v.at[c]],
                         rows_v.at[pl.ds(c * _CHUNK, _CHUNK)], sems.at[c])

    @pl.when(wid == 0)
    def _():
        pltpu.sync_copy(main_hbm, midx_v)
        pltpu.async_copy(table_hbm.at[midx_v], mrow_v, msem)

    _UNROLL = 8
    acc0 = tuple(jnp.zeros((_LANES,), jnp.float32) for _ in range(_NV))

    def chunk_body(c, a):
        pltpu.make_async_copy(table_hbm.at[idx_v.at[0]],
                              rows_v.at[pl.ds(0, _CHUNK)], sems.at[c]).wait()

        def body(i, a):
            r0 = c * _CHUNK + i * _UNROLL
            for u in range(_UNROLL):
                a = tuple(a[v] + rows_v[r0 + u, pl.ds(v * _LANES, _LANES)]
                          for v in range(_NV))
            return a

        return lax.fori_loop(0, _CHUNK // _UNROLL, body, a)

    acc = lax.fori_loop(0, _NCH, chunk_body, acc0)

    for v in range(_NV):
        acc_v[pl.ds(v * _LANES, _LANES)] = acc[v]
    pltpu.sync_copy(acc_v, partials_out.at[wid])

    @pl.when(wid == 0)
    def _():
        pltpu.make_async_copy(table_hbm.at[midx_v], mrow_v, msem).wait()
        pltpu.sync_copy(mrow_v, main_out)


def _tc_finish_body(partials_ref, mrow_ref, fcw_ref, fcb_ref, out_ref):
    s = jnp.sum(partials_ref[...], axis=0, keepdims=True)
    combined = s * (0.5 / _NTOK) + mrow_ref[...] * 0.5
    out_ref[...] = lax.dot_general(
        combined, fcw_ref[...], (((1,), (1,)), ((), ())),
        preferred_element_type=jnp.float32,
    ) + fcb_ref[...]


def kernel(ingredient_tokens, main_token, emb_table, fc_w, fc_b):
    partials, main_row = _sc_gather_sum(ingredient_tokens, main_token, emb_table)
    out = pl.pallas_call(
        _tc_finish_body,
        out_shape=jax.ShapeDtypeStruct((1, _DIM), jnp.float32),
    )(partials, main_row, fc_w, fc_b.reshape(1, _DIM))
    return out


# speedup vs baseline: 1.0525x; 1.0046x over previous
import functools

import jax
import jax.numpy as jnp
from jax import lax
from jax.experimental import pallas as pl
from jax.experimental.pallas import tpu as pltpu
from jax.experimental.pallas import tpu_sc as plsc

_VOCAB = 100000
_DIM = 128
_NTOK = 16384

_NC = 2
_NS = 16
_NW = _NC * _NS
_BPW = _NTOK // _NW
_CHUNK = 64
_NCH = _BPW // _CHUNK
_LANES = 16
_NV = _DIM // _LANES


@functools.partial(
    pl.kernel,
    out_type=[
        jax.ShapeDtypeStruct((_NW, _DIM), jnp.float32),
        jax.ShapeDtypeStruct((1, _DIM), jnp.float32),
    ],
    mesh=plsc.VectorSubcoreMesh(core_axis_name="c", subcore_axis_name="s"),
    scratch_types=[
        pltpu.VMEM((_NCH, _CHUNK), jnp.int32),
        pltpu.VMEM((_BPW, _DIM), jnp.float32),
        pltpu.VMEM((_DIM,), jnp.float32),
        pltpu.VMEM((1,), jnp.int32),
        pltpu.VMEM((1, _DIM), jnp.float32),
        pltpu.SemaphoreType.DMA((_NCH,)),
        pltpu.SemaphoreType.DMA((_NCH,)),
        pltpu.SemaphoreType.DMA,
    ],
)
def _sc_gather_sum(idx_hbm, main_hbm, table_hbm, partials_out, main_out,
                   idx_v, rows_v, acc_v, midx_v, mrow_v, isems, sems, msem):
    wid = lax.axis_index("s") * _NC + lax.axis_index("c")
    base = wid * _BPW
    for c in range(_NCH):
        pltpu.async_copy(idx_hbm.at[pl.ds(base + c * _CHUNK, _CHUNK)],
                         idx_v.at[c], isems.at[c])
    for c in range(_NCH):
        pltpu.make_async_copy(idx_hbm.at[pl.ds(base, _CHUNK)],
                              idx_v.at[c], isems.at[c]).wait()
        pltpu.async_copy(table_hbm.at[idx_v.at[c]],
                         rows_v.at[pl.ds(c * _CHUNK, _CHUNK)], sems.at[c])

    @pl.when(wid == 0)
    def _():
        pltpu.sync_copy(main_hbm, midx_v)
        pltpu.async_copy(table_hbm.at[midx_v], mrow_v, msem)

    _UNROLL = 8
    acc0 = tuple(jnp.zeros((_LANES,), jnp.float32) for _ in range(_NV))

    def chunk_body(c, a):
        pltpu.make_async_copy(table_hbm.at[idx_v.at[0]],
                              rows_v.at[pl.ds(0, _CHUNK)], sems.at[c]).wait()

        def body(r, a):
            return tuple(a[v] + rows_v[c * _CHUNK + r, pl.ds(v * _LANES, _LANES)]
                         for v in range(_NV))

        return plsc.parallel_loop(0, _CHUNK, 1, unroll=_UNROLL, carry=a)(body)

    acc = lax.fori_loop(0, _NCH, chunk_body, acc0)

    for v in range(_NV):
        acc_v[pl.ds(v * _LANES, _LANES)] = acc[v]
    pltpu.sync_copy(acc_v, partials_out.at[wid])

    @pl.when(wid == 0)
    def _():
        pltpu.make_async_copy(table_hbm.at[midx_v], mrow_v, msem).wait()
        pltpu.sync_copy(mrow_v, main_out)


def _tc_finish_body(partials_ref, mrow_ref, fcw_ref, fcb_ref, out_ref):
    s = jnp.sum(partials_ref[...], axis=0, keepdims=True)
    combined = s * (0.5 / _NTOK) + mrow_ref[...] * 0.5
    out_ref[...] = lax.dot_general(
        combined, fcw_ref[...], (((1,), (1,)), ((), ())),
        preferred_element_type=jnp.float32,
    ) + fcb_ref[...]


def kernel(ingredient_tokens, main_token, emb_table, fc_w, fc_b):
    partials, main_row = _sc_gather_sum(ingredient_tokens, main_token, emb_table)
    out = pl.pallas_call(
        _tc_finish_body,
        out_shape=jax.ShapeDtypeStruct((1, _DIM), jnp.float32),
    )(partials, main_row, fc_w, fc_b.reshape(1, _DIM))
    return out


# speedup vs baseline: 1.0594x; 1.0065x over previous
import functools

import jax
import jax.numpy as jnp
from jax import lax
from jax.experimental import pallas as pl
from jax.experimental.pallas import tpu as pltpu
from jax.experimental.pallas import tpu_sc as plsc

_VOCAB = 100000
_DIM = 128
_NTOK = 16384

_NC = 2
_NS = 16
_NW = _NC * _NS
_BPW = _NTOK // _NW
_CHUNK = 64
_NCH = _BPW // _CHUNK
_LANES = 16
_NV = _DIM // _LANES


@functools.partial(
    pl.kernel,
    out_type=[
        jax.ShapeDtypeStruct((_NW, _DIM), jnp.float32),
        jax.ShapeDtypeStruct((1, _DIM), jnp.float32),
    ],
    mesh=plsc.VectorSubcoreMesh(core_axis_name="c", subcore_axis_name="s"),
    scratch_types=[
        pltpu.VMEM((_NCH, _CHUNK), jnp.int32),
        pltpu.VMEM((_BPW, _DIM), jnp.float32),
        pltpu.VMEM((_DIM,), jnp.float32),
        pltpu.VMEM((1,), jnp.int32),
        pltpu.VMEM((1, _DIM), jnp.float32),
        pltpu.SemaphoreType.DMA((_NCH,)),
        pltpu.SemaphoreType.DMA((_NCH,)),
        pltpu.SemaphoreType.DMA,
    ],
)
def _sc_gather_sum(idx_hbm, main_hbm, table_hbm, partials_out, main_out,
                   idx_v, rows_v, acc_v, midx_v, mrow_v, isems, sems, msem):
    wid = lax.axis_index("s") * _NC + lax.axis_index("c")
    base = wid * _BPW
    for c in range(_NCH):
        pltpu.async_copy(idx_hbm.at[pl.ds(base + c * _CHUNK, _CHUNK)],
                         idx_v.at[c], isems.at[c])
    for c in range(_NCH):
        pltpu.make_async_copy(idx_hbm.at[pl.ds(base, _CHUNK)],
                              idx_v.at[c], isems.at[c]).wait()
        pltpu.async_copy(table_hbm.at[idx_v.at[c]],
                         rows_v.at[pl.ds(c * _CHUNK, _CHUNK)], sems.at[c])

    @pl.when(wid == 0)
    def _():
        pltpu.sync_copy(main_hbm, midx_v)
        pltpu.async_copy(table_hbm.at[midx_v], mrow_v, msem)

    _UNROLL = 16
    acc0 = tuple(jnp.zeros((_LANES,), jnp.float32) for _ in range(_NV))

    def chunk_body(c, a):
        pltpu.make_async_copy(table_hbm.at[idx_v.at[0]],
                              rows_v.at[pl.ds(0, _CHUNK)], sems.at[c]).wait()

        def body(r, a):
            return tuple(a[v] + rows_v[c * _CHUNK + r, pl.ds(v * _LANES, _LANES)]
                         for v in range(_NV))

        return plsc.parallel_loop(0, _CHUNK, 1, unroll=_UNROLL, carry=a)(body)

    acc = lax.fori_loop(0, _NCH, chunk_body, acc0)

    for v in range(_NV):
        acc_v[pl.ds(v * _LANES, _LANES)] = acc[v]
    pltpu.sync_copy(acc_v, partials_out.at[wid])

    @pl.when(wid == 0)
    def _():
        pltpu.make_async_copy(table_hbm.at[midx_v], mrow_v, msem).wait()
        pltpu.sync_copy(mrow_v, main_out)


def _tc_finish_body(partials_ref, mrow_ref, fcw_ref, fcb_ref, out_ref):
    s = jnp.sum(partials_ref[...], axis=0, keepdims=True)
    combined = s * (0.5 / _NTOK) + mrow_ref[...] * 0.5
    out_ref[...] = lax.dot_general(
        combined, fcw_ref[...], (((1,), (1,)), ((), ())),
        preferred_element_type=jnp.float32,
    ) + fcb_ref[...]


def kernel(ingredient_tokens, main_token, emb_table, fc_w, fc_b):
    partials, main_row = _sc_gather_sum(ingredient_tokens, main_token, emb_table)
    out = pl.pallas_call(
        _tc_finish_body,
        out_shape=jax.ShapeDtypeStruct((1, _DIM), jnp.float32),
    )(partials, main_row, fc_w, fc_b.reshape(1, _DIM))
    return out


# speedup vs baseline: 1.0668x; 1.0070x over previous
import functools

import jax
import jax.numpy as jnp
from jax import lax
from jax.experimental import pallas as pl
from jax.experimental.pallas import tpu as pltpu
from jax.experimental.pallas import tpu_sc as plsc

_VOCAB = 100000
_DIM = 128
_NTOK = 16384

_NC = 2
_NS = 16
_NW = _NC * _NS
_BPW = _NTOK // _NW
_CHUNK = 64
_NCH = _BPW // _CHUNK
_LANES = 16
_NV = _DIM // _LANES


@functools.partial(
    pl.kernel,
    out_type=[
        jax.ShapeDtypeStruct((_NW, _DIM), jnp.float32),
        jax.ShapeDtypeStruct((1, _DIM), jnp.float32),
    ],
    mesh=plsc.VectorSubcoreMesh(core_axis_name="c", subcore_axis_name="s"),
    scratch_types=[
        pltpu.VMEM((_NCH, _CHUNK), jnp.int32),
        pltpu.VMEM((_BPW, _DIM), jnp.float32),
        pltpu.VMEM((_DIM,), jnp.float32),
        pltpu.VMEM((1,), jnp.int32),
        pltpu.VMEM((1, _DIM), jnp.float32),
        pltpu.SemaphoreType.DMA((_NCH,)),
        pltpu.SemaphoreType.DMA((_NCH,)),
        pltpu.SemaphoreType.DMA,
    ],
)
def _sc_gather_sum(idx_hbm, main_hbm, table_hbm, partials_out, main_out,
                   idx_v, rows_v, acc_v, midx_v, mrow_v, isems, sems, msem):
    wid = lax.axis_index("s") * _NC + lax.axis_index("c")
    base = wid * _BPW
    @pl.when(wid == 0)
    def _():
        pltpu.async_copy(main_hbm, midx_v, msem)

    for c in range(_NCH):
        pltpu.async_copy(idx_hbm.at[pl.ds(base + c * _CHUNK, _CHUNK)],
                         idx_v.at[c], isems.at[c])
    for c in range(_NCH):
        pltpu.make_async_copy(idx_hbm.at[pl.ds(base, _CHUNK)],
                              idx_v.at[c], isems.at[c]).wait()
        pltpu.async_copy(table_hbm.at[idx_v.at[c]],
                         rows_v.at[pl.ds(c * _CHUNK, _CHUNK)], sems.at[c])

    @pl.when(wid == 0)
    def _():
        pltpu.make_async_copy(main_hbm, midx_v, msem).wait()
        pltpu.async_copy(table_hbm.at[midx_v], mrow_v, msem)

    _UNROLL = 16
    acc0 = tuple(jnp.zeros((_LANES,), jnp.float32) for _ in range(_NV))

    def chunk_body(c, a):
        pltpu.make_async_copy(table_hbm.at[idx_v.at[0]],
                              rows_v.at[pl.ds(0, _CHUNK)], sems.at[c]).wait()

        def body(r, a):
            return tuple(a[v] + rows_v[c * _CHUNK + r, pl.ds(v * _LANES, _LANES)]
                         for v in range(_NV))

        return plsc.parallel_loop(0, _CHUNK, 1, unroll=_UNROLL, carry=a)(body)

    acc = lax.fori_loop(0, _NCH, chunk_body, acc0)

    for v in range(_NV):
        acc_v[pl.ds(v * _LANES, _LANES)] = acc[v]

    @pl.when(wid == 0)
    def _():
        pltpu.make_async_copy(table_hbm.at[midx_v], mrow_v, msem).wait()
        pltpu.async_copy(mrow_v, main_out, msem)
    pltpu.sync_copy(acc_v, partials_out.at[wid])
    @pl.when(wid == 0)
    def _():
        pltpu.make_async_copy(mrow_v, main_out, msem).wait()


def _tc_finish_body(partials_ref, mrow_ref, fcw_ref, fcb_ref, out_ref):
    s = jnp.sum(partials_ref[...], axis=0, keepdims=True)
    combined = s * (0.5 / _NTOK) + mrow_ref[...] * 0.5
    out_ref[...] = lax.dot_general(
        combined, fcw_ref[...], (((1,), (1,)), ((), ())),
        preferred_element_type=jnp.float32,
    ) + fcb_ref[...]


def kernel(ingredient_tokens, main_token, emb_table, fc_w, fc_b):
    partials, main_row = _sc_gather_sum(ingredient_tokens, main_token, emb_table)
    out = pl.pallas_call(
        _tc_finish_body,
        out_shape=jax.ShapeDtypeStruct((1, _DIM), jnp.float32),
    )(partials, main_row, fc_w, fc_b.reshape(1, _DIM))
    return out
